# baseline (speedup 1.0000x reference)
import jax
import jax.numpy as jnp
import numpy as np
from jax import lax
from jax.experimental import pallas as pl
from jax.experimental.pallas import tpu as pltpu
from jax.experimental.pallas import tpu_sc as plsc

B, S, H = 4, 2048, 768
N = B * S
EPS = 1e-12

NC, NS, L = 2, 16, 16
NW = NC * NS
TOK_PER_W = N // NW
CHUNK = 8
NCHUNK = TOK_PER_W // CHUNK
NSLOT = 4
NQUAD = NCHUNK // NSLOT
HV = H // L
HV2 = HV // 2
TYPES, MAXPOS, MAXSEG, MAXSPK = 2, 2048, 64, 16
SEG_OFF = MAXPOS
SPK_OFF = MAXPOS + MAXSEG
TYPE_OFF = MAXPOS + MAXSEG + MAXSPK
NROWS = MAXPOS + MAXSEG + MAXSPK + TYPES

_PERM = (np.arange(H).reshape(HV2, 2, L).transpose(0, 2, 1).reshape(-1))


def _rsqrt(x):
    xh = 0.5 * x
    i = lax.bitcast_convert_type(x, jnp.int32)
    i = jnp.int32(0x5F3759DF) - (i >> 1)
    y = lax.bitcast_convert_type(i, jnp.float32)
    y = y * (1.5 - xh * y * y)
    y = y * (1.5 - xh * y * y)
    y = y * (1.5 - xh * y * y)
    return y


def _body(ids_w, ids_c, word_hbm, combo_hbm, gamma_hbm, beta_hbm, out_hbm,
          iw, ic, gbuf, bbuf,
          bw0, bw1, bw2, bw3, bc0, bc1, bc2, bc3, ob0, ob1, ob2, ob3,
          semg0, semg1, semg2, semg3, semo0, semo1, semo2, semo3, sems):
    sid = lax.axis_index("s")
    wid = sid * NC + lax.axis_index("c")
    rbase = wid * NCHUNK

    staged = ((gamma_hbm, gbuf), (beta_hbm, bbuf),
              (ids_w.at[pl.ds(rbase, NCHUNK)], iw),
              (ids_c.at[pl.ds(rbase, NCHUNK)], ic))
    for src, dst in staged:
        pltpu.async_copy(src, dst, sems)
    for src, dst in staged:
        pltpu.make_async_copy(src, dst, sems).wait()

    def fire(c, bw, bc, semg):
        pltpu.async_copy(word_hbm.at[iw.at[c]], bw, semg)
        pltpu.async_copy(combo_hbm.at[ic.at[c]], bc, semg)

    def drain(c, bw, bc, semg):
        pltpu.make_async_copy(word_hbm.at[iw.at[c]], bw, semg).wait()
        pltpu.make_async_copy(combo_hbm.at[ic.at[c]], bc, semg).wait()

    slots = ((bw0, bc0, ob0, semg0, semo0),
             (bw1, bc1, ob1, semg1, semo1),
             (bw2, bc2, ob2, semg2, semo2),
             (bw3, bc3, ob3, semg3, semo3))

    for k, (bw, bc, ob, semg, semo) in enumerate(slots):
        fire(k, bw, bc, semg)

    def compute_chunk(bw, bc, ob):
        def row_body(r, carry):
            s = jnp.zeros((L,), jnp.float32)
            ss = jnp.zeros((L,), jnp.float32)
            for j2 in range(HV2):
                bsl = pl.ds(j2 * L, L)
                fmt = plsc.PackFormat.INTERLEAVED
                bf = jnp.bfloat16
                pa, pb = plsc.unpack(plsc.bitcast(bc[r, bsl], bf),
                                     format=fmt)
                ga, gb = plsc.unpack(plsc.bitcast(bc[r + CHUNK, bsl], bf),
                                     format=fmt)
                ka, kb = plsc.unpack(
                    plsc.bitcast(bc[r + 2 * CHUNK, bsl], bf), format=fmt)
                ta, tb = plsc.unpack(
                    plsc.bitcast(bc[r + 3 * CHUNK, bsl], bf), format=fmt)
                hsa = pl.ds(j2 * 2 * L, L)
                hsb = pl.ds(j2 * 2 * L + L, L)
                va = bw[r, hsa] + pa + ga + ka + ta
                vb = bw[r, hsb] + pb + gb + kb + tb
                ob[r, hsa] = va
                ob[r, hsb] = vb
                s = s + va + vb
                ss = ss + va * va + vb * vb
            mean = lax.reduce_sum_p.bind(s, axes=(0,)) * (1.0 / H)
            msq = lax.reduce_sum_p.bind(ss, axes=(0,)) * (1.0 / H)
            rstd = _rsqrt(msq - mean * mean + EPS)
            for j in range(HV):
                hs = pl.ds(j * L, L)
                ob[r, hs] = (ob[r, hs] - mean) * rstd * gbuf[hs] + bbuf[hs]
            return carry
        lax.fori_loop(0, CHUNK, row_body, 0)

    def quad_body(i, carry):
        for k, (bw, bc, ob, semg, semo) in enumerate(slots):
            c = NSLOT * i + k
            osl = pl.ds((rbase + c) * CHUNK, CHUNK)
            drain(c, bw, bc, semg)

            @pl.when(i > 0)
            def _():
                pltpu.make_async_copy(ob, out_hbm.at[osl], semo).wait()

            compute_chunk(bw, bc, ob)
            pltpu.async_copy(ob, out_hbm.at[osl], semo)

            @pl.when(i < NQUAD - 1)
            def _():
                fire(c + NSLOT, bw, bc, semg)
        return carry

    lax.fori_loop(0, NQUAD, quad_body, 0)

    for k, (bw, bc, ob, semg, semo) in enumerate(slots):
        c = NCHUNK - NSLOT + k
        pltpu.make_async_copy(
            ob, out_hbm.at[pl.ds((rbase + c) * CHUNK, CHUNK)], semo).wait()


@jax.jit
def _run(ids_w, ids_c, word_emb, combo_emb, ln_gamma, ln_beta):
    mesh = plsc.VectorSubcoreMesh(core_axis_name="c", subcore_axis_name="s",
                                  num_cores=NC, num_subcores=NS)
    f = pl.kernel(
        _body,
        out_type=jax.ShapeDtypeStruct((N, H), jnp.float32),
        mesh=mesh,
        scratch_types=(
            [pltpu.VMEM((NCHUNK, CHUNK), jnp.int32),
             pltpu.VMEM((NCHUNK, 4 * CHUNK), jnp.int32),
             pltpu.VMEM((H,), jnp.float32),
             pltpu.VMEM((H,), jnp.float32)]
            + [pltpu.VMEM((CHUNK, H), jnp.float32)] * 4
            + [pltpu.VMEM((4 * CHUNK, H // 2), jnp.int32)] * 4
            + [pltpu.VMEM((CHUNK, H), jnp.float32)] * 4
            + [pltpu.SemaphoreType.DMA] * 9),
        compiler_params=pltpu.CompilerParams(needs_layout_passes=False),
        name="wsw_embed_ln",
    )
    return f(ids_w, ids_c, word_emb, combo_emb, ln_gamma, ln_beta)


def kernel(input_ids, token_type_ids, position_ids, segment_ids, speaker_ids,
           word_emb, type_emb, pos_emb, seg_emb, spk_emb, ln_gamma, ln_beta):
    ids_w = input_ids.reshape(N // CHUNK, CHUNK).astype(jnp.int32)
    combo = jnp.stack(
        [position_ids.reshape(N // CHUNK, CHUNK).astype(jnp.int32),
         segment_ids.reshape(N // CHUNK, CHUNK).astype(jnp.int32) + SEG_OFF,
         speaker_ids.reshape(N // CHUNK, CHUNK).astype(jnp.int32) + SPK_OFF,
         token_type_ids.reshape(N // CHUNK, CHUNK).astype(jnp.int32)
         + TYPE_OFF],
        axis=1).reshape(N // CHUNK, 4 * CHUNK)
    combo_emb = lax.bitcast_convert_type(
        jnp.concatenate([pos_emb, seg_emb, spk_emb, type_emb], axis=0)
        [:, _PERM].astype(jnp.bfloat16).reshape(NROWS, H // 2, 2),
        jnp.int32)
    out = _run(ids_w, combo, word_emb, combo_emb, ln_gamma, ln_beta)
    return out.reshape(B, S, H)


# speedup vs baseline: 1.5882x; 1.5882x over previous
import jax
import jax.numpy as jnp
import numpy as np
from jax import lax
from jax.experimental import pallas as pl
from jax.experimental.pallas import tpu as pltpu
from jax.experimental.pallas import tpu_sc as plsc

B, S, H = 4, 2048, 768
N = B * S
EPS = 1e-12

NC, NS, L = 2, 16, 16
NW = NC * NS
TOK_PER_W = N // NW
CHUNK = 8
NCHUNK = TOK_PER_W // CHUNK
NSLOT = 4
NQUAD = NCHUNK // NSLOT
HV = H // L
HV2 = HV // 2
HW = H // 2
TYPES, MAXPOS, MAXSEG, MAXSPK = 2, 2048, 64, 16
SPK_OFF = MAXSEG
NROWS = MAXSEG + MAXSPK

_PERM = (np.arange(H).reshape(HV2, 2, L).transpose(0, 2, 1).reshape(-1))


def _rsqrt(x):
    xh = 0.5 * x
    i = lax.bitcast_convert_type(x, jnp.int32)
    i = jnp.int32(0x5F3759DF) - (i >> 1)
    y = lax.bitcast_convert_type(i, jnp.float32)
    y = y * (1.5 - xh * y * y)
    y = y * (1.5 - xh * y * y)
    y = y * (1.5 - xh * y * y)
    return y


def _body(ids_w, ids_p, tgk_hbm, word_hbm, pos_hbm, combo_hbm,
          gamma_hbm, beta_hbm, out_hbm,
          iw, ip, tgk, tbl, gbuf, bbuf,
          bw0, bw1, bw2, bw3, bp0, bp1, bp2, bp3,
          semg0, semg1, semg2, semg3, semo0, semo1, semo2, semo3, sems):
    sid = lax.axis_index("s")
    wid = sid * NC + lax.axis_index("c")
    rbase = wid * NCHUNK
    tbase = wid * TOK_PER_W

    staged = ((gamma_hbm, gbuf), (beta_hbm, bbuf), (combo_hbm, tbl),
              (ids_w.at[pl.ds(rbase, NCHUNK)], iw),
              (ids_p.at[pl.ds(rbase, NCHUNK)], ip),
              (tgk_hbm.at[pl.ds(tbase, TOK_PER_W)], tgk))
    for src, dst in staged:
        pltpu.async_copy(src, dst, sems)
    for src, dst in staged:
        pltpu.make_async_copy(src, dst, sems).wait()

    def fire_w(c, bw, semg):
        pltpu.async_copy(word_hbm.at[iw.at[c]], bw, semg)

    def fire_p(c, bp, semg):
        pltpu.async_copy(pos_hbm.at[ip.at[c]], bp, semg)

    def drain(c, bw, bp, semg):
        pltpu.make_async_copy(word_hbm.at[iw.at[c]], bw, semg).wait()
        pltpu.make_async_copy(pos_hbm.at[ip.at[c]], bp, semg).wait()

    slots = ((bw0, bp0, semg0, semo0),
             (bw1, bp1, semg1, semo1),
             (bw2, bp2, semg2, semo2),
             (bw3, bp3, semg3, semo3))

    for k, (bw, bp, semg, semo) in enumerate(slots):
        fire_w(k, bw, semg)
    fire_p(0, bp0, semg0)
    fire_p(1, bp1, semg1)

    fmt = plsc.PackFormat.INTERLEAVED
    bf = jnp.bfloat16

    def compute_chunk(c, bw, bp, ob):
        def row_body(r, carry):
            tok = c * CHUNK + r
            gi, ki = plsc.unpack(plsc.bitcast(tgk[tok, :], jnp.int16),
                                 format=fmt)
            s = jnp.zeros((L,), jnp.float32)
            ss = jnp.zeros((L,), jnp.float32)
            for j2 in range(HV2):
                ga, gb = plsc.unpack(
                    plsc.bitcast(plsc.load_gather(tbl, [gi]), bf), format=fmt)
                ka, kb = plsc.unpack(
                    plsc.bitcast(plsc.load_gather(tbl, [ki]), bf), format=fmt)
                gi = gi + L
                ki = ki + L
                hsa = pl.ds(j2 * 2 * L, L)
                hsb = pl.ds(j2 * 2 * L + L, L)
                va = bw[r, hsa] + bp[r, hsa] + ga + ka
                vb = bw[r, hsb] + bp[r, hsb] + gb + kb
                ob[r, hsa] = va
                ob[r, hsb] = vb
                s = s + va + vb
                ss = ss + va * va + vb * vb
            mean = lax.reduce_sum_p.bind(s, axes=(0,)) * (1.0 / H)
            msq = lax.reduce_sum_p.bind(ss, axes=(0,)) * (1.0 / H)
            rstd = _rsqrt(msq - mean * mean + EPS)
            for j in range(HV):
                hs = pl.ds(j * L, L)
                ob[r, hs] = (ob[r, hs] - mean) * rstd * gbuf[hs] + bbuf[hs]
            return carry
        lax.fori_loop(0, CHUNK, row_body, 0)

    def quad_body(i, carry):
        for k, (bw, bp, semg, semo) in enumerate(slots):
            c = NSLOT * i + k
            osl = pl.ds((rbase + c) * CHUNK, CHUNK)
            drain(c, bw, bp, semg)
            compute_chunk(c, bw, bp, bp)
            pltpu.async_copy(bp, out_hbm.at[osl], semo)

            @pl.when(i < NQUAD - 1)
            def _():
                fire_w(c + NSLOT, bw, semg)

            bp2s = slots[(k + 2) % NSLOT][1]
            semo2s = slots[(k + 2) % NSLOT][3]
            if k < 2:
                @pl.when(i > 0)
                def _():
                    pltpu.make_async_copy(bp2s, out_hbm.at[osl],
                                          semo2s).wait()
                fire_p(c + 2, bp2s, slots[(k + 2) % NSLOT][2])
            else:
                pltpu.make_async_copy(bp2s, out_hbm.at[osl], semo2s).wait()

                @pl.when(i < NQUAD - 1)
                def _():
                    fire_p(c + 2, bp2s, slots[(k + 2) % NSLOT][2])
        return carry

    lax.fori_loop(0, NQUAD, quad_body, 0)

    for k in (2, 3):
        bp, semo = slots[k][1], slots[k][3]
        c = NCHUNK - NSLOT + k
        pltpu.make_async_copy(
            bp, out_hbm.at[pl.ds((rbase + c) * CHUNK, CHUNK)], semo).wait()


@jax.jit
def _run(ids_w, ids_p, tgk, word_emb, pos_emb, combo_emb,
         ln_gamma, ln_beta):
    mesh = plsc.VectorSubcoreMesh(core_axis_name="c", subcore_axis_name="s",
                                  num_cores=NC, num_subcores=NS)
    f = pl.kernel(
        _body,
        out_type=jax.ShapeDtypeStruct((N, H), jnp.float32),
        mesh=mesh,
        scratch_types=(
            [pltpu.VMEM((NCHUNK, CHUNK), jnp.int32),
             pltpu.VMEM((NCHUNK, CHUNK), jnp.int32),
             pltpu.VMEM((TOK_PER_W, L), jnp.int32),
             pltpu.VMEM((NROWS * HW,), jnp.int32),
             pltpu.VMEM((H,), jnp.float32),
             pltpu.VMEM((H,), jnp.float32)]
            + [pltpu.VMEM((CHUNK, H), jnp.float32)] * 4
            + [pltpu.VMEM((CHUNK, H), jnp.float32)] * 4
            + [pltpu.SemaphoreType.DMA] * 9),
        compiler_params=pltpu.CompilerParams(needs_layout_passes=False),
        name="wsw_embed_ln",
    )
    return f(ids_w, ids_p, tgk, word_emb, pos_emb, combo_emb,
             ln_gamma, ln_beta)


def kernel(input_ids, token_type_ids, position_ids, segment_ids, speaker_ids,
           word_emb, type_emb, pos_emb, seg_emb, spk_emb, ln_gamma, ln_beta):
    ids_w = input_ids.reshape(N // CHUNK, CHUNK).astype(jnp.int32)
    ids_p = (position_ids.reshape(-1).astype(jnp.int32)
             + token_type_ids.reshape(-1).astype(jnp.int32) * MAXPOS
             ).reshape(N // CHUNK, CHUNK)
    pos_aug = jnp.concatenate(
        [pos_emb + type_emb[0][None, :], pos_emb + type_emb[1][None, :]],
        axis=0)
    lanes = jnp.arange(L, dtype=jnp.int16)
    gi16 = ((segment_ids.reshape(-1).astype(jnp.int16) * HW)[:, None]
            + lanes)
    ki16 = (((speaker_ids.reshape(-1).astype(jnp.int16) + SPK_OFF)
             * HW)[:, None] + lanes)
    tgk = lax.bitcast_convert_type(
        jnp.stack([gi16, ki16], axis=2).reshape(N, L, 2), jnp.int32)
    combo_emb = lax.bitcast_convert_type(
        jnp.concatenate([seg_emb, spk_emb], axis=0)
        [:, _PERM].astype(jnp.bfloat16).reshape(NROWS * HW, 2),
        jnp.int32)
    out = _run(ids_w, ids_p, tgk, word_emb, pos_aug, combo_emb,
               ln_gamma, ln_beta)
    return out.reshape(B, S, H)
